# baseline (speedup 1.0000x reference)
import functools

import jax
import jax.numpy as jnp
from jax import lax
from jax.experimental import pallas as pl
from jax.experimental.pallas import tpu as pltpu
from jax.experimental.pallas import tpu_sc as plsc

V_N = 1000
D_N = 128
W_N = D_N // 2
B_N = 16384
C_N = 20

NC = 2
NS = 16
NW = NC * NS
LANES = 16
GRP = LANES
FLUSH_G = 16
STAGE_COLS = FLUSH_G * GRP

B_SC = 8192
B_CH = B_SC
B_TC = B_N - B_SC
V_PAD = 1024
ROWS_W = B_CH // NW
NGRP = ROWS_W // GRP


def _sc_pool_body(xt_hbm, u_hbm, out_hbm, tab_v, idx_v, stage_v, sem):
    wid = lax.axis_index("s") * NC + lax.axis_index("c")
    row_base = pl.multiple_of(wid * ROWS_W, ROWS_W)
    pltpu.async_copy(u_hbm, tab_v, sem)
    for c in range(C_N):
        pltpu.async_copy(
            xt_hbm.at[pl.ds(c * B_CH + row_base, ROWS_W)],
            idx_v.at[pl.ds(c * ROWS_W, ROWS_W)],
            sem,
        )
    pltpu.make_async_copy(u_hbm, tab_v, sem).wait()
    for c in range(C_N):
        pltpu.make_async_copy(
            xt_hbm.at[pl.ds(c * B_CH + row_base, ROWS_W)],
            idx_v.at[pl.ds(c * ROWS_W, ROWS_W)],
            sem,
        ).wait()

    def grp_body(g, _):
        idxs = [idx_v[pl.ds(c * ROWS_W + g * GRP, GRP)] for c in range(C_N)]
        col = pl.ds((g % FLUSH_G) * GRP, GRP)
        for k in range(W_N):
            kv = jnp.full((LANES,), k * V_N, jnp.int32)
            acc0 = plsc.bitcast(
                plsc.load_gather(tab_v, [idxs[0] + kv]), jnp.bfloat16)
            acc1 = plsc.bitcast(
                plsc.load_gather(tab_v, [idxs[1] + kv]), jnp.bfloat16)
            for c in range(2, C_N, 2):
                acc0 = acc0 + plsc.bitcast(
                    plsc.load_gather(tab_v, [idxs[c] + kv]), jnp.bfloat16)
                acc1 = acc1 + plsc.bitcast(
                    plsc.load_gather(tab_v, [idxs[c + 1] + kv]), jnp.bfloat16)
            lo, hi = plsc.unpack(acc0 + acc1, format=plsc.PackFormat.INTERLEAVED)
            stage_v[2 * k, col] = lo
            stage_v[2 * k + 1, col] = hi

        @pl.when((g + 1) % FLUSH_G == 0)
        def _():
            flush_col = pl.multiple_of(
                row_base + (g + 1 - FLUSH_G) * GRP, STAGE_COLS)
            pltpu.sync_copy(stage_v, out_hbm.at[:, pl.ds(flush_col, STAGE_COLS)])

        return 0

    lax.fori_loop(0, NGRP, grp_body, 0)


def _sc_pool(xt_chunk, u_packed_t):
    mesh = plsc.VectorSubcoreMesh(
        core_axis_name="c", subcore_axis_name="s", num_cores=NC, num_subcores=NS
    )
    fn = pl.kernel(
        _sc_pool_body,
        out_type=jax.ShapeDtypeStruct((D_N, B_CH), jnp.float32),
        mesh=mesh,
        compiler_params=pltpu.CompilerParams(needs_layout_passes=False),
        scratch_types=[
            pltpu.VMEM((W_N * V_N,), jnp.int32),
            pltpu.VMEM((C_N * ROWS_W,), jnp.int32),
            pltpu.VMEM((D_N, STAGE_COLS), jnp.float32),
            pltpu.SemaphoreType.DMA,
        ],
    )
    return fn(xt_chunk, u_packed_t)


TB = 2048
N1 = B_SC // TB


def _tc_proj_body(l1_ref, x2_ref, u_ref, w_ref, b_ref, o_ref):
    i = pl.program_id(0)

    @pl.when(i < N1)
    def _():
        acc = lax.dot_general(
            l1_ref[...], w_ref[...],
            (((0,), (1,)), ((), ())),
            preferred_element_type=jnp.float32,
        )
        o_ref[...] = acc * (1.0 / C_N) + b_ref[...]

    @pl.when(i >= N1)
    def _():
        vio = lax.broadcasted_iota(jnp.int32, (TB, V_PAD), 1)
        cnt = jnp.zeros((TB, V_PAD), jnp.float32)
        for c in range(C_N):
            xc = x2_ref[:, c].reshape(TB, 1)
            cnt = cnt + (xc == vio).astype(jnp.float32)
        l1b = lax.dot_general(
            cnt, u_ref[...],
            (((1,), (0,)), ((), ())),
            preferred_element_type=jnp.float32,
        )
        acc = lax.dot_general(
            l1b, w_ref[...],
            (((1,), (1,)), ((), ())),
            preferred_element_type=jnp.float32,
        )
        o_ref[...] = acc * (1.0 / C_N) + b_ref[...]


def _tc_proj(l1t, x2, u_pad, v_w, v_b2d):
    return pl.pallas_call(
        _tc_proj_body,
        grid=(B_N // TB,),
        in_specs=[
            pl.BlockSpec((D_N, TB), lambda i: (0, jnp.minimum(i, N1 - 1))),
            pl.BlockSpec((TB, C_N), lambda i: (jnp.maximum(i - N1, 0), 0)),
            pl.BlockSpec((V_PAD, D_N), lambda i: (0, 0)),
            pl.BlockSpec((V_N, D_N), lambda i: (0, 0)),
            pl.BlockSpec((1, V_N), lambda i: (0, 0)),
        ],
        out_specs=pl.BlockSpec((TB, V_N), lambda i: (i, 0)),
        out_shape=jax.ShapeDtypeStruct((B_N, V_N), jnp.float32),
    )(l1t, x2, u_pad, v_w, v_b2d)


@jax.jit
def kernel(x, U, V_w, V_b):
    xi = x.astype(jnp.int32)
    xt = xi[:B_SC].T
    x2 = xi[B_SC:]
    u_packed_t = lax.bitcast_convert_type(
        U.astype(jnp.bfloat16).reshape(V_N, W_N, 2), jnp.int32
    ).T.reshape(-1)
    u_pad = jnp.pad(U, ((0, V_PAD - V_N), (0, 0)))
    v_b2d = V_b.reshape(1, V_N)
    l1t = _sc_pool(xt.reshape(-1), u_packed_t)
    return _tc_proj(l1t, x2, u_pad, V_w, v_b2d)


# speedup vs baseline: 1.0982x; 1.0982x over previous
import functools

import jax
import jax.numpy as jnp
from jax import lax
from jax.experimental import pallas as pl
from jax.experimental.pallas import tpu as pltpu
from jax.experimental.pallas import tpu_sc as plsc

V_N = 1000
D_N = 128
W_N = D_N // 2
B_N = 16384
C_N = 20

NC = 2
NS = 16
NW = NC * NS
LANES = 16
GRP = LANES
FLUSH_G = 16
STAGE_COLS = FLUSH_G * GRP

N_CHUNK = 1
B_CH = B_N // N_CHUNK
ROWS_W = B_CH // NW
NGRP = ROWS_W // GRP


def _sc_pool_body(xt_hbm, u_hbm, out_hbm, tab_v, idx_v, stage_v, sem):
    wid = lax.axis_index("s") * NC + lax.axis_index("c")
    row_base = pl.multiple_of(wid * ROWS_W, ROWS_W)
    pltpu.async_copy(u_hbm, tab_v, sem)
    for c in range(C_N):
        pltpu.async_copy(
            xt_hbm.at[pl.ds(c * B_CH + row_base, ROWS_W)],
            idx_v.at[pl.ds(c * ROWS_W, ROWS_W)],
            sem,
        )
    pltpu.make_async_copy(u_hbm, tab_v, sem).wait()
    for c in range(C_N):
        pltpu.make_async_copy(
            xt_hbm.at[pl.ds(c * B_CH + row_base, ROWS_W)],
            idx_v.at[pl.ds(c * ROWS_W, ROWS_W)],
            sem,
        ).wait()

    def grp_body(g, _):
        idxs = [idx_v[pl.ds(c * ROWS_W + g * GRP, GRP)] for c in range(C_N)]
        col = pl.ds((g % FLUSH_G) * GRP, GRP)
        for k in range(W_N):
            kv = jnp.full((LANES,), k * V_N, jnp.int32)
            acc0 = plsc.bitcast(
                plsc.load_gather(tab_v, [idxs[0] + kv]), jnp.bfloat16)
            acc1 = plsc.bitcast(
                plsc.load_gather(tab_v, [idxs[1] + kv]), jnp.bfloat16)
            for c in range(2, C_N, 2):
                acc0 = acc0 + plsc.bitcast(
                    plsc.load_gather(tab_v, [idxs[c] + kv]), jnp.bfloat16)
                acc1 = acc1 + plsc.bitcast(
                    plsc.load_gather(tab_v, [idxs[c + 1] + kv]), jnp.bfloat16)
            lo, hi = plsc.unpack(acc0 + acc1, format=plsc.PackFormat.INTERLEAVED)
            stage_v[2 * k, col] = lo
            stage_v[2 * k + 1, col] = hi

        @pl.when((g + 1) % FLUSH_G == 0)
        def _():
            flush_col = pl.multiple_of(
                row_base + (g + 1 - FLUSH_G) * GRP, STAGE_COLS)
            pltpu.sync_copy(stage_v, out_hbm.at[:, pl.ds(flush_col, STAGE_COLS)])

        return 0

    lax.fori_loop(0, NGRP, grp_body, 0)


def _sc_pool(xt_chunk, u_packed_t):
    mesh = plsc.VectorSubcoreMesh(
        core_axis_name="c", subcore_axis_name="s", num_cores=NC, num_subcores=NS
    )
    fn = pl.kernel(
        _sc_pool_body,
        out_type=jax.ShapeDtypeStruct((D_N, B_CH), jnp.float32),
        mesh=mesh,
        compiler_params=pltpu.CompilerParams(needs_layout_passes=False),
        scratch_types=[
            pltpu.VMEM((W_N * V_N,), jnp.int32),
            pltpu.VMEM((C_N * ROWS_W,), jnp.int32),
            pltpu.VMEM((D_N, STAGE_COLS), jnp.float32),
            pltpu.SemaphoreType.DMA,
        ],
    )
    return fn(xt_chunk, u_packed_t)


def _tc_proj_body(l1_ref, w_ref, b_ref, o_ref):
    acc = lax.dot_general(
        l1_ref[...], w_ref[...],
        (((0,), (1,)), ((), ())),
        preferred_element_type=jnp.float32,
    )
    o_ref[...] = acc * (1.0 / C_N) + b_ref[...]


def _tc_proj(l1t, v_w, v_b2d):
    tb = 2048
    return pl.pallas_call(
        _tc_proj_body,
        grid=(B_CH // tb,),
        in_specs=[
            pl.BlockSpec((D_N, tb), lambda i: (0, i)),
            pl.BlockSpec((V_N, D_N), lambda i: (0, 0)),
            pl.BlockSpec((1, V_N), lambda i: (0, 0)),
        ],
        out_specs=pl.BlockSpec((tb, V_N), lambda i: (i, 0)),
        out_shape=jax.ShapeDtypeStruct((B_CH, V_N), jnp.float32),
    )(l1t, v_w, v_b2d)


@jax.jit
def kernel(x, U, V_w, V_b):
    xt = x.T.astype(jnp.int32)
    u_packed_t = lax.bitcast_convert_type(
        U.astype(jnp.bfloat16).reshape(V_N, W_N, 2), jnp.int32
    ).T.reshape(-1)
    v_b2d = V_b.reshape(1, V_N)
    l1t = _sc_pool(xt.reshape(-1), u_packed_t)
    return _tc_proj(l1t, V_w, v_b2d)
